# Initial kernel scaffold; baseline (speedup 1.0000x reference)
import functools

import jax
import jax.numpy as jnp
from jax import lax
from jax.experimental import pallas as pl
from jax.experimental.pallas import tpu as pltpu
from jax.experimental.pallas import tpu_sc as plsc

N = 10000
E = 320000
D = 128
NLAYERS = 3
NSUB = 16
NCORE = 2
DH = D // NCORE
NPAD = 10240
RPT = NPAD // NSUB
EPT = E // NSUB
EB = 128
NBLK = EPT // EB
ETAIL = EPT - NBLK * EB
CPG = DH // 16


def _rsqrt16(x):
    i = plsc.bitcast(x, jnp.int32)
    i = jnp.int32(0x5F3759DF) - lax.shift_right_arithmetic(i, jnp.int32(1))
    y = plsc.bitcast(i, jnp.float32)
    for _ in range(3):
        y = y * (jnp.float32(1.5) - jnp.float32(0.5) * x * y * y)
    return jnp.where(x > jnp.float32(0.0), y, jnp.float32(0.0))


def _body(x_hbm, ei_hbm, out_hbm, h_hbm,
          dego_s, degi_s, acc_s,
          idx_v, idx2_v, tidx_v, tidx2_v, ones_v, tones_v,
          rows_v, a16, f16, h16, z16, zrow_v, dl_v, ln_v, rn_v, rln_v, sem):
    sc = lax.axis_index("c")
    sid = lax.axis_index("s")
    rp0 = sid * RPT
    e0 = sid * EPT
    dbase = sc * DH
    hbase = sc * N
    hoffv = jnp.full((16,), hbase, jnp.int32)
    nch = jnp.where(sid == NSUB - 1, (N - (NSUB - 1) * RPT) // 16, RPT // 16)

    zero16 = jnp.zeros((16,), jnp.float32)
    one16 = jnp.ones((16,), jnp.float32)
    for rr in range(16):
        for c in range(CPG):
            z16[rr, pl.ds(c * 16, 16)] = zero16
    for k in range(RPT // 16):
        zrow_v[pl.ds(k * 16, 16)] = zero16
    for k in range(EB // 16):
        ones_v[pl.ds(k * 16, 16)] = one16
    for k in range(ETAIL // 16):
        tones_v[pl.ds(k * 16, 16)] = one16

    pltpu.sync_copy(zrow_v, dego_s.at[pl.ds(rp0, RPT)])
    pltpu.sync_copy(zrow_v, degi_s.at[pl.ds(rp0, RPT)])
    plsc.subcore_barrier()

    def deg_blk(b, carry):
        eb = e0 + b * EB
        pltpu.sync_copy(ei_hbm.at[0, pl.ds(eb, EB)], idx_v)
        pltpu.sync_copy(ei_hbm.at[1, pl.ds(eb, EB)], idx2_v)
        pltpu.sync_copy(ones_v, dego_s.at[idx_v], add=True)
        pltpu.sync_copy(ones_v, degi_s.at[idx2_v], add=True)
        return carry

    lax.fori_loop(0, NBLK, deg_blk, 0)
    ebt = e0 + NBLK * EB
    pltpu.sync_copy(ei_hbm.at[0, pl.ds(ebt, ETAIL)], tidx_v)
    pltpu.sync_copy(ei_hbm.at[1, pl.ds(ebt, ETAIL)], tidx2_v)
    pltpu.sync_copy(tones_v, dego_s.at[tidx_v], add=True)
    pltpu.sync_copy(tones_v, degi_s.at[tidx2_v], add=True)
    plsc.subcore_barrier()

    pltpu.sync_copy(dego_s.at[pl.ds(rp0, RPT)], dl_v)

    def lnorm(k, carry):
        s = pl.ds(k * 16, 16)
        ln_v[s] = _rsqrt16(dl_v[s])
        return carry

    lax.fori_loop(0, RPT // 16, lnorm, 0)
    pltpu.sync_copy(degi_s.at[pl.ds(rp0, RPT)], dl_v)

    def rnorm(k, carry):
        s = pl.ds(k * 16, 16)
        rv = _rsqrt16(dl_v[s])
        rn_v[s] = rv
        rln_v[s] = rv * ln_v[s]
        return carry

    lax.fori_loop(0, RPT // 16, rnorm, 0)

    def prep_chunk(k, carry):
        rr0 = rp0 + k * 16
        pltpu.sync_copy(x_hbm.at[pl.ds(rr0, 16), pl.ds(dbase, DH)], a16)
        for rr in range(16):
            iv = jnp.full((16,), k * 16 + rr, jnp.int32)
            lnv = plsc.load_gather(ln_v, [iv])
            for c in range(CPG):
                s = pl.ds(c * 16, 16)
                h16[rr, s] = a16[rr, s] * lnv
        pltpu.sync_copy(a16, out_hbm.at[0, pl.ds(rr0, 16), pl.ds(dbase, DH)])
        pltpu.sync_copy(h16, h_hbm.at[pl.ds(hbase + rr0, 16)])
        return carry

    lax.fori_loop(0, nch, prep_chunk, 0)
    plsc.subcore_barrier()

    for l in range(NLAYERS):
        def zchunk(k, carry):
            pltpu.sync_copy(z16, acc_s.at[pl.ds(rp0 + k * 16, 16)])
            return carry

        lax.fori_loop(0, RPT // 16, zchunk, 0)
        plsc.subcore_barrier()

        def edge_blk(b, carry):
            eb = e0 + b * EB
            pltpu.sync_copy(ei_hbm.at[0, pl.ds(eb, EB)], idx_v)
            pltpu.sync_copy(ei_hbm.at[1, pl.ds(eb, EB)], idx2_v)
            for k in range(EB // 16):
                s = pl.ds(k * 16, 16)
                idx_v[s] = idx_v[s] + hoffv
            pltpu.async_copy(h_hbm.at[idx_v], rows_v, sem).wait()
            pltpu.sync_copy(rows_v, acc_s.at[idx2_v], add=True)
            return carry

        lax.fori_loop(0, NBLK, edge_blk, 0)
        pltpu.sync_copy(ei_hbm.at[0, pl.ds(ebt, ETAIL)], tidx_v)
        pltpu.sync_copy(ei_hbm.at[1, pl.ds(ebt, ETAIL)], tidx2_v)
        for k in range(ETAIL // 16):
            s = pl.ds(k * 16, 16)
            tidx_v[s] = tidx_v[s] + hoffv
        pltpu.async_copy(h_hbm.at[tidx_v], rows_v.at[pl.ds(0, ETAIL)], sem).wait()
        pltpu.sync_copy(rows_v.at[pl.ds(0, ETAIL)], acc_s.at[tidx2_v], add=True)
        plsc.subcore_barrier()

        def scale_chunk(k, carry, l=l):
            rr0 = rp0 + k * 16
            pltpu.sync_copy(acc_s.at[pl.ds(rr0, 16)], a16)
            for rr in range(16):
                iv = jnp.full((16,), k * 16 + rr, jnp.int32)
                rnv = plsc.load_gather(rn_v, [iv])
                rlnv = plsc.load_gather(rln_v, [iv])
                for c in range(CPG):
                    s = pl.ds(c * 16, 16)
                    v = a16[rr, s]
                    f16[rr, s] = v * rnv
                    h16[rr, s] = v * rlnv
            pltpu.sync_copy(
                f16, out_hbm.at[l + 1, pl.ds(rr0, 16), pl.ds(dbase, DH)])
            if l < NLAYERS - 1:
                pltpu.sync_copy(h16, h_hbm.at[pl.ds(hbase + rr0, 16)])
            return carry

        lax.fori_loop(0, nch, scale_chunk, 0)
        plsc.subcore_barrier()


@functools.partial(
    pl.kernel,
    out_type=[
        jax.ShapeDtypeStruct((NLAYERS + 1, N, D), jnp.float32),
        jax.ShapeDtypeStruct((NCORE * N, DH), jnp.float32),
    ],
    mesh=plsc.VectorSubcoreMesh(core_axis_name="c", subcore_axis_name="s"),
    scratch_types=[
        pltpu.VMEM_SHARED((NPAD,), jnp.float32),
        pltpu.VMEM_SHARED((NPAD,), jnp.float32),
        pltpu.VMEM_SHARED((NPAD, DH), jnp.float32),
        pltpu.VMEM((EB,), jnp.int32),
        pltpu.VMEM((EB,), jnp.int32),
        pltpu.VMEM((ETAIL,), jnp.int32),
        pltpu.VMEM((ETAIL,), jnp.int32),
        pltpu.VMEM((EB,), jnp.float32),
        pltpu.VMEM((ETAIL,), jnp.float32),
        pltpu.VMEM((EB, DH), jnp.float32),
        pltpu.VMEM((16, DH), jnp.float32),
        pltpu.VMEM((16, DH), jnp.float32),
        pltpu.VMEM((16, DH), jnp.float32),
        pltpu.VMEM((16, DH), jnp.float32),
        pltpu.VMEM((RPT,), jnp.float32),
        pltpu.VMEM((RPT,), jnp.float32),
        pltpu.VMEM((RPT,), jnp.float32),
        pltpu.VMEM((RPT,), jnp.float32),
        pltpu.VMEM((RPT,), jnp.float32),
        pltpu.SemaphoreType.DMA,
    ],
)
def _gcn(x_hbm, ei_hbm, out_hbm, h_hbm, *scratch):
    _body(x_hbm, ei_hbm, out_hbm, h_hbm, *scratch)


def kernel(x, edge_index):
    out, _ = _gcn(x, edge_index)
    return out


# speedup vs baseline: 3.7104x; 3.7104x over previous
import functools

import jax
import jax.numpy as jnp
from jax import lax
from jax.experimental import pallas as pl
from jax.experimental.pallas import tpu as pltpu
from jax.experimental.pallas import tpu_sc as plsc

N = 10000
E = 320000
D = 128
NLAYERS = 3
NSUB = 16
NCORE = 2
DH = D // NCORE
NPAD = 10240
RPT = NPAD // NSUB
EPT = E // NSUB
EB = 128
NBLK = EPT // EB
ETAIL = EPT - NBLK * EB
CPG = DH // 16


def _rsqrt16(x):
    i = plsc.bitcast(x, jnp.int32)
    i = jnp.int32(0x5F3759DF) - lax.shift_right_arithmetic(i, jnp.int32(1))
    y = plsc.bitcast(i, jnp.float32)
    for _ in range(3):
        y = y * (jnp.float32(1.5) - jnp.float32(0.5) * x * y * y)
    return jnp.where(x > jnp.float32(0.0), y, jnp.float32(0.0))


def _body(x_hbm, src_hbm, dst_hbm, out_hbm, h_hbm,
          dego_s, degi_s, acc_s,
          idx_v, idx2_v, tidx_v, tidx2_v, ones_v, tones_v,
          rows_v, a16, f16, h16, z16, zrow_v, dl_v, ln_v, rn_v, rln_v, sem):
    sc = lax.axis_index("c")
    sid = lax.axis_index("s")
    rp0 = sid * RPT
    e0 = sid * EPT
    dbase = sc * DH
    hbase = sc * N
    hoffv = jnp.full((16,), hbase, jnp.int32)
    nch = jnp.where(sid == NSUB - 1, (N - (NSUB - 1) * RPT) // 16, RPT // 16)

    zero16 = jnp.zeros((16,), jnp.float32)
    one16 = jnp.ones((16,), jnp.float32)
    for rr in range(16):
        for c in range(CPG):
            z16[rr, pl.ds(c * 16, 16)] = zero16
    for k in range(RPT // 16):
        zrow_v[pl.ds(k * 16, 16)] = zero16
    for k in range(EB // 16):
        ones_v[pl.ds(k * 16, 16)] = one16
    for k in range(ETAIL // 16):
        tones_v[pl.ds(k * 16, 16)] = one16

    pltpu.sync_copy(zrow_v, dego_s.at[pl.ds(rp0, RPT)])
    pltpu.sync_copy(zrow_v, degi_s.at[pl.ds(rp0, RPT)])
    plsc.subcore_barrier()

    def deg_blk(b, carry):
        eb = e0 + b * EB
        pltpu.sync_copy(src_hbm.at[pl.ds(eb, EB)], idx_v)
        pltpu.sync_copy(dst_hbm.at[pl.ds(eb, EB)], idx2_v)
        pltpu.sync_copy(ones_v, dego_s.at[idx_v], add=True)
        pltpu.sync_copy(ones_v, degi_s.at[idx2_v], add=True)
        return carry

    lax.fori_loop(0, NBLK, deg_blk, 0)
    ebt = e0 + NBLK * EB
    pltpu.sync_copy(src_hbm.at[pl.ds(ebt, ETAIL)], tidx_v)
    pltpu.sync_copy(dst_hbm.at[pl.ds(ebt, ETAIL)], tidx2_v)
    pltpu.sync_copy(tones_v, dego_s.at[tidx_v], add=True)
    pltpu.sync_copy(tones_v, degi_s.at[tidx2_v], add=True)
    plsc.subcore_barrier()

    pltpu.sync_copy(dego_s.at[pl.ds(rp0, RPT)], dl_v)

    def lnorm(k, carry):
        s = pl.ds(k * 16, 16)
        ln_v[s] = _rsqrt16(dl_v[s])
        return carry

    lax.fori_loop(0, RPT // 16, lnorm, 0)
    pltpu.sync_copy(degi_s.at[pl.ds(rp0, RPT)], dl_v)

    def rnorm(k, carry):
        s = pl.ds(k * 16, 16)
        rv = _rsqrt16(dl_v[s])
        rn_v[s] = rv
        rln_v[s] = rv * ln_v[s]
        return carry

    lax.fori_loop(0, RPT // 16, rnorm, 0)

    def prep_chunk(k, carry):
        rr0 = rp0 + k * 16
        pltpu.sync_copy(x_hbm.at[pl.ds(rr0, 16), pl.ds(dbase, DH)], a16)
        for rr in range(16):
            iv = jnp.full((16,), k * 16 + rr, jnp.int32)
            lnv = plsc.load_gather(ln_v, [iv])
            for c in range(CPG):
                s = pl.ds(c * 16, 16)
                h16[rr, s] = a16[rr, s] * lnv
        pltpu.sync_copy(a16, out_hbm.at[0, pl.ds(rr0, 16), pl.ds(dbase, DH)])
        pltpu.sync_copy(h16, h_hbm.at[pl.ds(hbase + rr0, 16)])
        return carry

    lax.fori_loop(0, nch, prep_chunk, 0)
    plsc.subcore_barrier()

    for l in range(NLAYERS):
        def zchunk(k, carry):
            pltpu.sync_copy(z16, acc_s.at[pl.ds(rp0 + k * 16, 16)])
            return carry

        lax.fori_loop(0, RPT // 16, zchunk, 0)
        plsc.subcore_barrier()

        def edge_blk(b, carry):
            eb = e0 + b * EB
            pltpu.sync_copy(src_hbm.at[pl.ds(eb, EB)], idx_v)
            pltpu.sync_copy(dst_hbm.at[pl.ds(eb, EB)], idx2_v)
            for k in range(EB // 16):
                s = pl.ds(k * 16, 16)
                idx_v[s] = idx_v[s] + hoffv
            pltpu.async_copy(h_hbm.at[idx_v], rows_v, sem).wait()
            pltpu.sync_copy(rows_v, acc_s.at[idx2_v], add=True)
            return carry

        lax.fori_loop(0, NBLK, edge_blk, 0)
        pltpu.sync_copy(src_hbm.at[pl.ds(ebt, ETAIL)], tidx_v)
        pltpu.sync_copy(dst_hbm.at[pl.ds(ebt, ETAIL)], tidx2_v)
        for k in range(ETAIL // 16):
            s = pl.ds(k * 16, 16)
            tidx_v[s] = tidx_v[s] + hoffv
        pltpu.async_copy(h_hbm.at[tidx_v], rows_v.at[pl.ds(0, ETAIL)], sem).wait()
        pltpu.sync_copy(rows_v.at[pl.ds(0, ETAIL)], acc_s.at[tidx2_v], add=True)
        plsc.subcore_barrier()

        def scale_chunk(k, carry, l=l):
            rr0 = rp0 + k * 16
            pltpu.sync_copy(acc_s.at[pl.ds(rr0, 16)], a16)
            for rr in range(16):
                iv = jnp.full((16,), k * 16 + rr, jnp.int32)
                rnv = plsc.load_gather(rn_v, [iv])
                rlnv = plsc.load_gather(rln_v, [iv])
                for c in range(CPG):
                    s = pl.ds(c * 16, 16)
                    v = a16[rr, s]
                    f16[rr, s] = v * rnv
                    h16[rr, s] = v * rlnv
            pltpu.sync_copy(
                f16, out_hbm.at[l + 1, pl.ds(rr0, 16), pl.ds(dbase, DH)])
            if l < NLAYERS - 1:
                pltpu.sync_copy(h16, h_hbm.at[pl.ds(hbase + rr0, 16)])
            return carry

        lax.fori_loop(0, nch, scale_chunk, 0)
        plsc.subcore_barrier()


@functools.partial(
    pl.kernel,
    out_type=[
        jax.ShapeDtypeStruct((NLAYERS + 1, N, D), jnp.float32),
        jax.ShapeDtypeStruct((NCORE * N, DH), jnp.float32),
    ],
    mesh=plsc.VectorSubcoreMesh(core_axis_name="c", subcore_axis_name="s"),
    compiler_params=pltpu.CompilerParams(use_tc_tiling_on_sc=False,
                                        needs_layout_passes=False),
    scratch_types=[
        pltpu.VMEM_SHARED((NPAD,), jnp.float32),
        pltpu.VMEM_SHARED((NPAD,), jnp.float32),
        pltpu.VMEM_SHARED((NPAD, DH), jnp.float32),
        pltpu.VMEM((EB,), jnp.int32),
        pltpu.VMEM((EB,), jnp.int32),
        pltpu.VMEM((ETAIL,), jnp.int32),
        pltpu.VMEM((ETAIL,), jnp.int32),
        pltpu.VMEM((EB,), jnp.float32),
        pltpu.VMEM((ETAIL,), jnp.float32),
        pltpu.VMEM((EB, DH), jnp.float32),
        pltpu.VMEM((16, DH), jnp.float32),
        pltpu.VMEM((16, DH), jnp.float32),
        pltpu.VMEM((16, DH), jnp.float32),
        pltpu.VMEM((16, DH), jnp.float32),
        pltpu.VMEM((RPT,), jnp.float32),
        pltpu.VMEM((RPT,), jnp.float32),
        pltpu.VMEM((RPT,), jnp.float32),
        pltpu.VMEM((RPT,), jnp.float32),
        pltpu.VMEM((RPT,), jnp.float32),
        pltpu.SemaphoreType.DMA,
    ],
)
def _gcn(x_hbm, src_hbm, dst_hbm, out_hbm, h_hbm, *scratch):
    _body(x_hbm, src_hbm, dst_hbm, out_hbm, h_hbm, *scratch)


def kernel(x, edge_index):
    out, _ = _gcn(x, edge_index[0], edge_index[1])
    return out


# speedup vs baseline: 5.8968x; 1.5893x over previous
import functools

import jax
import jax.numpy as jnp
from jax import lax
from jax.experimental import pallas as pl
from jax.experimental.pallas import tpu as pltpu
from jax.experimental.pallas import tpu_sc as plsc

N = 10000
E = 320000
D = 128
NLAYERS = 3
NSUB = 16
NCORE = 2
DH = D // NCORE
NPAD = 10240
RPT = NPAD // NSUB
EPT = E // NSUB
EB = 128
NBLK = EPT // EB
ETAIL = EPT - NBLK * EB
NBP = 158
EPAD = NBP * EB
DROW = 10200
CPG = DH // 16


def _rsqrt16(x):
    i = plsc.bitcast(x, jnp.int32)
    i = jnp.int32(0x5F3759DF) - lax.shift_right_arithmetic(i, jnp.int32(1))
    y = plsc.bitcast(i, jnp.float32)
    for _ in range(3):
        y = y * (jnp.float32(1.5) - jnp.float32(0.5) * x * y * y)
    return jnp.where(x > jnp.float32(0.0), y, jnp.float32(0.0))


def _body(x_hbm, src_hbm, dst_hbm, out_hbm, h_hbm,
          dego_s, degi_s, acc_s,
          idx_v, idx2_v, tidx_v, tidx2_v, ones_v, tones_v,
          sidx_v, didx_v, bidx_v, rows0, rows1,
          a16, f16, h16, z16, zrow_v, dl_v, ln_v, rn_v, rln_v,
          gsem0, gsem1):
    sc = lax.axis_index("c")
    sid = lax.axis_index("s")
    rp0 = sid * RPT
    e0 = sid * EPT
    dbase = sc * DH
    hbase = sc * N
    hoffv = jnp.full((16,), hbase, jnp.int32)
    nch = jnp.where(sid == NSUB - 1, (N - (NSUB - 1) * RPT) // 16, RPT // 16)

    zero16 = jnp.zeros((16,), jnp.float32)
    one16 = jnp.ones((16,), jnp.float32)
    for rr in range(16):
        for c in range(CPG):
            z16[rr, pl.ds(c * 16, 16)] = zero16
    for k in range(RPT // 16):
        zrow_v[pl.ds(k * 16, 16)] = zero16
    for k in range(EB // 16):
        ones_v[pl.ds(k * 16, 16)] = one16
    for k in range(ETAIL // 16):
        tones_v[pl.ds(k * 16, 16)] = one16

    pltpu.sync_copy(zrow_v, dego_s.at[pl.ds(rp0, RPT)])
    pltpu.sync_copy(zrow_v, degi_s.at[pl.ds(rp0, RPT)])
    plsc.subcore_barrier()

    def deg_blk(b, carry):
        eb = e0 + b * EB
        pltpu.sync_copy(src_hbm.at[pl.ds(eb, EB)], idx_v)
        pltpu.sync_copy(dst_hbm.at[pl.ds(eb, EB)], idx2_v)
        pltpu.sync_copy(ones_v, dego_s.at[idx_v], add=True)
        pltpu.sync_copy(ones_v, degi_s.at[idx2_v], add=True)
        return carry

    lax.fori_loop(0, NBLK, deg_blk, 0)
    ebt = e0 + NBLK * EB
    pltpu.sync_copy(src_hbm.at[pl.ds(ebt, ETAIL)], tidx_v)
    pltpu.sync_copy(dst_hbm.at[pl.ds(ebt, ETAIL)], tidx2_v)
    pltpu.sync_copy(tones_v, dego_s.at[tidx_v], add=True)
    pltpu.sync_copy(tones_v, degi_s.at[tidx2_v], add=True)
    plsc.subcore_barrier()

    pltpu.sync_copy(src_hbm.at[pl.ds(e0, EPT)], sidx_v.at[pl.ds(0, EPT)])
    pltpu.sync_copy(dst_hbm.at[pl.ds(e0, EPT)], didx_v.at[pl.ds(0, EPT)])
    zi16 = jnp.zeros((16,), jnp.int32)
    drow16 = jnp.full((16,), DROW, jnp.int32)
    for k in range(EPT // 16, EPAD // 16):
        sidx_v[pl.ds(k * 16, 16)] = zi16
        didx_v[pl.ds(k * 16, 16)] = drow16

    def offs(k, carry):
        sl = pl.ds(k * 16, 16)
        sidx_v[sl] = sidx_v[sl] + hoffv
        return carry

    lax.fori_loop(0, EPAD // 16, offs, 0)

    pltpu.sync_copy(dego_s.at[pl.ds(rp0, RPT)], dl_v)

    def lnorm(k, carry):
        s = pl.ds(k * 16, 16)
        ln_v[s] = _rsqrt16(dl_v[s])
        return carry

    lax.fori_loop(0, RPT // 16, lnorm, 0)
    pltpu.sync_copy(degi_s.at[pl.ds(rp0, RPT)], dl_v)

    def rnorm(k, carry):
        s = pl.ds(k * 16, 16)
        rv = _rsqrt16(dl_v[s])
        rn_v[s] = rv
        rln_v[s] = rv * ln_v[s]
        return carry

    lax.fori_loop(0, RPT // 16, rnorm, 0)

    def prep_chunk(k, carry):
        rr0 = rp0 + k * 16
        pltpu.sync_copy(x_hbm.at[pl.ds(rr0, 16), pl.ds(dbase, DH)], a16)
        for rr in range(16):
            iv = jnp.full((16,), k * 16 + rr, jnp.int32)
            lnv = plsc.load_gather(ln_v, [iv])
            for c in range(CPG):
                s = pl.ds(c * 16, 16)
                h16[rr, s] = a16[rr, s] * lnv
        pltpu.sync_copy(a16, out_hbm.at[0, pl.ds(rr0, 16), pl.ds(dbase, DH)])
        pltpu.sync_copy(h16, h_hbm.at[pl.ds(hbase + rr0, 16)])
        return carry

    lax.fori_loop(0, nch, prep_chunk, 0)
    plsc.subcore_barrier()

    for l in range(NLAYERS):
        def zchunk(k, carry):
            pltpu.sync_copy(z16, acc_s.at[pl.ds(rp0 + k * 16, 16)])
            return carry

        lax.fori_loop(0, RPT // 16, zchunk, 0)
        plsc.subcore_barrier()

        def g_issue(bb, rbuf, gsem):
            pltpu.async_copy(
                h_hbm.at[sidx_v.at[pl.ds(bb * EB, EB)]], rbuf, gsem)

        def g_wait(bb, rbuf, gsem):
            pltpu.make_async_copy(
                h_hbm.at[sidx_v.at[pl.ds(bb * EB, EB)]], rbuf, gsem).wait()

        g_issue(0, rows0, gsem0)
        g_issue(1, rows1, gsem1)

        def edge_pair(i, carry):
            b = i * 2
            for j, (rbuf, gsem) in enumerate(((rows0, gsem0), (rows1, gsem1))):
                bb = b + j
                g_wait(bb, rbuf, gsem)
                for k in range(EB // 16):
                    bidx_v[pl.ds(k * 16, 16)] = didx_v[pl.ds(bb * EB + k * 16, 16)]
                pltpu.sync_copy(rbuf, acc_s.at[bidx_v], add=True)

                @pl.when(bb + 2 < NBP)
                def _issue(bb=bb, rbuf=rbuf, gsem=gsem):
                    g_issue(bb + 2, rbuf, gsem)

            return carry

        lax.fori_loop(0, NBP // 2, edge_pair, 0)
        plsc.subcore_barrier()

        def scale_chunk(k, carry, l=l):
            rr0 = rp0 + k * 16
            pltpu.sync_copy(acc_s.at[pl.ds(rr0, 16)], a16)
            for rr in range(16):
                iv = jnp.full((16,), k * 16 + rr, jnp.int32)
                rnv = plsc.load_gather(rn_v, [iv])
                rlnv = plsc.load_gather(rln_v, [iv])
                for c in range(CPG):
                    s = pl.ds(c * 16, 16)
                    v = a16[rr, s]
                    f16[rr, s] = v * rnv
                    h16[rr, s] = v * rlnv
            pltpu.sync_copy(
                f16, out_hbm.at[l + 1, pl.ds(rr0, 16), pl.ds(dbase, DH)])
            if l < NLAYERS - 1:
                pltpu.sync_copy(h16, h_hbm.at[pl.ds(hbase + rr0, 16)])
            return carry

        lax.fori_loop(0, nch, scale_chunk, 0)
        plsc.subcore_barrier()


@functools.partial(
    pl.kernel,
    out_type=[
        jax.ShapeDtypeStruct((NLAYERS + 1, N, D), jnp.float32),
        jax.ShapeDtypeStruct((NCORE * N, DH), jnp.float32),
    ],
    mesh=plsc.VectorSubcoreMesh(core_axis_name="c", subcore_axis_name="s"),
    compiler_params=pltpu.CompilerParams(use_tc_tiling_on_sc=False,
                                        needs_layout_passes=False),
    scratch_types=[
        pltpu.VMEM_SHARED((NPAD,), jnp.float32),
        pltpu.VMEM_SHARED((NPAD,), jnp.float32),
        pltpu.VMEM_SHARED((NPAD, DH), jnp.float32),
        pltpu.VMEM((EB,), jnp.int32),
        pltpu.VMEM((EB,), jnp.int32),
        pltpu.VMEM((ETAIL,), jnp.int32),
        pltpu.VMEM((ETAIL,), jnp.int32),
        pltpu.VMEM((EB,), jnp.float32),
        pltpu.VMEM((ETAIL,), jnp.float32),
        pltpu.VMEM((EPAD,), jnp.int32),
        pltpu.VMEM((EPAD,), jnp.int32),
        pltpu.VMEM((EB,), jnp.int32),
        pltpu.VMEM((EB, DH), jnp.float32),
        pltpu.VMEM((EB, DH), jnp.float32),
        pltpu.VMEM((16, DH), jnp.float32),
        pltpu.VMEM((16, DH), jnp.float32),
        pltpu.VMEM((16, DH), jnp.float32),
        pltpu.VMEM((16, DH), jnp.float32),
        pltpu.VMEM((RPT,), jnp.float32),
        pltpu.VMEM((RPT,), jnp.float32),
        pltpu.VMEM((RPT,), jnp.float32),
        pltpu.VMEM((RPT,), jnp.float32),
        pltpu.VMEM((RPT,), jnp.float32),
        pltpu.SemaphoreType.DMA,
        pltpu.SemaphoreType.DMA,
    ],
)
def _gcn(x_hbm, src_hbm, dst_hbm, out_hbm, h_hbm, *scratch):
    _body(x_hbm, src_hbm, dst_hbm, out_hbm, h_hbm, *scratch)


def kernel(x, edge_index):
    out, _ = _gcn(x, edge_index[0], edge_index[1])
    return out
